# Initial kernel scaffold; baseline (speedup 1.0000x reference)
import jax
import jax.numpy as jnp
from jax.experimental import pallas as pl


def kernel(anchor, target):
    raise NotImplementedError("write your pallas kernel here")


# speedup vs baseline: 1.4687x; 1.4687x over previous
import functools

import numpy as np
import jax
import jax.numpy as jnp
from jax import lax
from jax.experimental import pallas as pl
from jax.experimental.pallas import tpu as pltpu
from jax.experimental.pallas import tpu_sc as plsc

_SAMPLE = 20
_CHUNK = 256
_NBUF = 3


def _threefry2x32(k0, k1, x0, x1):
    x0 = np.asarray(x0, np.uint32).copy()
    x1 = np.asarray(x1, np.uint32).copy()
    k0 = np.uint32(k0)
    k1 = np.uint32(k1)
    ks = [k0, k1, np.uint32(k0 ^ k1 ^ np.uint32(0x1BD11BDA))]
    rot = [(13, 15, 26, 6), (17, 29, 16, 24)]
    x0 = (x0 + ks[0]).astype(np.uint32)
    x1 = (x1 + ks[1]).astype(np.uint32)
    for i in range(5):
        for r in rot[i % 2]:
            x0 = (x0 + x1).astype(np.uint32)
            x1 = ((x1 << np.uint32(r)) | (x1 >> np.uint32(32 - r))).astype(np.uint32)
            x1 = x0 ^ x1
        x0 = (x0 + ks[(i + 1) % 3]).astype(np.uint32)
        x1 = (x1 + ks[(i + 2) % 3] + np.uint32(i + 1)).astype(np.uint32)
    return x0, x1


def _np_randint_key42(n: int, maxval: int) -> np.ndarray:
    s1, s2 = _threefry2x32(0, 42, np.zeros(2, np.uint32), np.arange(2, dtype=np.uint32))
    zero = np.zeros(n, np.uint32)
    iota = np.arange(n, dtype=np.uint32)
    h1, h2 = _threefry2x32(s1[0], s2[0], zero, iota)
    l1, l2 = _threefry2x32(s1[1], s2[1], zero, iota)
    hi, lo = h1 ^ h2, l1 ^ l2
    span = np.uint32(maxval)
    m = np.uint32(np.uint32(65536) % span)
    mult = np.uint32(np.uint32(m * m) % span)
    off = ((hi % span).astype(np.uint32) * mult + (lo % span)) % span
    return off.astype(np.int32)


@functools.lru_cache(maxsize=None)
def _work_indices(bs: int, nw: int):
    ridx = _np_randint_key42(_SAMPLE * bs, bs).astype(np.int64)

    pos_rows = 2 * bs
    p = np.arange(pos_rows, dtype=np.int64)
    pos_t = np.where(p % 2 == 0, p // 2, bs + p // 2)

    neg_rows = 2 * _SAMPLE * bs
    q = np.arange(neg_rows, dtype=np.int64)
    neg_t = np.where(q % 2 == 0, (q // 2) // _SAMPLE, bs + ridx[q // 2])

    pos_items = pos_rows // nw // _CHUNK
    neg_items = neg_rows // nw // _CHUNK
    n_items = pos_items + neg_items
    n_pad = -n_items % 8
    widx = np.concatenate(
        [
            pos_t.reshape(nw, pos_items, _CHUNK),
            neg_t.reshape(nw, neg_items, _CHUNK),
            np.zeros((nw, n_pad, _CHUNK), np.int64),
        ],
        axis=1,
    ).astype(np.int32)
    return jnp.asarray(widx.reshape(nw * (n_items + n_pad) * _CHUNK))


@functools.lru_cache(maxsize=None)
def _build_kernel(bs: int):
    info = plsc.get_sparse_core_info()
    nw = info.num_cores * info.num_subcores

    pos_rows = 2 * bs
    neg_rows = 2 * _SAMPLE * bs
    pos_per_w = pos_rows // nw
    neg_per_w = neg_rows // nw
    pos_items = pos_per_w // _CHUNK
    neg_items = neg_per_w // _CHUNK
    n_items = pos_items + neg_items
    n_padded = n_items + (-n_items % 8)

    mesh = plsc.VectorSubcoreMesh(core_axis_name="c", subcore_axis_name="s")

    @functools.partial(
        pl.kernel,
        mesh=mesh,
        out_type=(
            jax.ShapeDtypeStruct((pos_rows, 64), jnp.float32),
            jax.ShapeDtypeStruct((neg_rows, 64), jnp.float32),
        ),
        scratch_types=[
            pltpu.VMEM((n_padded * _CHUNK,), jnp.int32),
            pltpu.VMEM((_CHUNK, 64), jnp.float32),
            pltpu.VMEM((_CHUNK, 64), jnp.float32),
            pltpu.VMEM((_CHUNK, 64), jnp.float32),
            pltpu.SemaphoreType.DMA,
            pltpu.SemaphoreType.DMA,
            pltpu.SemaphoreType.DMA,
        ],
        compiler_params=pltpu.CompilerParams(use_tc_tiling_on_sc=False),
    )
    def k(table_hbm, widx_hbm, pos_out, neg_out, idx_v, b0, b1, b2, s0, s1, s2):
        wid = lax.axis_index("s") * info.num_cores + lax.axis_index("c")
        bufs = (b0, b1, b2)
        sems = (s0, s1, s2)

        nw_idx = n_padded * _CHUNK
        pltpu.sync_copy(widx_hbm.at[pl.ds(wid * nw_idx, nw_idx)], idx_v)

        def start_gather(item):
            j = item % _NBUF
            return pltpu.async_copy(
                table_hbm.at[idx_v.at[pl.ds(item * _CHUNK, _CHUNK)]], bufs[j], sems[j]
            )

        def out_slice(item):
            if item < pos_items:
                return pos_out.at[pl.ds(wid * pos_per_w + item * _CHUNK, _CHUNK)]
            i = item - pos_items
            return neg_out.at[pl.ds(wid * neg_per_w + i * _CHUNK, _CHUNK)]

        handles = [start_gather(i) for i in range(_NBUF)]
        for item in range(n_items):
            j = item % _NBUF
            handles[j].wait()
            pltpu.sync_copy(bufs[j], out_slice(item))
            if item + _NBUF < n_items:
                handles[j] = start_gather(item + _NBUF)

    return k


def kernel(anchor, target):
    bs, d = target.shape
    info = plsc.get_sparse_core_info()
    nw = info.num_cores * info.num_subcores
    table = jnp.concatenate([anchor, target], axis=0)
    widx = _work_indices(bs, nw)
    pos_flat, neg_flat = _build_kernel(bs)(table, widx)
    pos = pos_flat.reshape(bs, 2, d)
    neg = neg_flat.reshape(_SAMPLE * bs, 2, d)
    return pos, neg
